# baseline (speedup 1.0000x reference)
import functools
import jax
import jax.numpy as jnp
from jax import lax
from jax.experimental import pallas as pl
from jax.experimental.pallas import tpu as pltpu
from jax.experimental.pallas import tpu_sc as plsc

CB = 8


def _make_sc_probe(B, C, H, W, nbatch):
    NW = 32
    cpw = (nbatch * C) // NW
    HH = H // 2

    mesh = plsc.VectorSubcoreMesh(core_axis_name="c", subcore_axis_name="s")

    @functools.partial(
        pl.kernel,
        mesh=mesh,
        out_type=jax.ShapeDtypeStruct((nbatch, C, H, W), jnp.float32),
        scratch_types=[
            pltpu.VMEM((HH, W), jnp.float32),
            pltpu.VMEM((HH, W), jnp.float32),
        ],
    )
    def sc_probe(x_hbm, n_hbm, o_hbm, xb, nb):
        cid = lax.axis_index("c")
        sid = lax.axis_index("s")
        wid = cid * 16 + sid
        g0 = wid * cpw

        def body(k2, _):
            g = g0 + k2 // 2
            h0 = (k2 % 2) * HH
            b = (B - nbatch) + g // C
            c = g % C
            pltpu.sync_copy(x_hbm.at[b, c, pl.ds(h0, HH), :], xb)
            pltpu.sync_copy(n_hbm.at[b, c, pl.ds(h0, HH), :], nb)
            pltpu.sync_copy(xb, o_hbm.at[b - (B - nbatch), c, pl.ds(h0, HH), :])
            return 0

        lax.fori_loop(0, cpw * 2, body, 0)

    return sc_probe


def _qsample_body(t_ref, c1tab_ref, c2tab_ref, x_ref, n_ref, o_ref):
    b = pl.program_id(0)
    tt = t_ref[b]
    c1 = c1tab_ref[tt]
    c2 = c2tab_ref[tt]
    o_ref[...] = x_ref[...] * c1 + n_ref[...] * c2


def kernel(x_start, t, noise, sqrt_alphas_cumprod, sqrt_one_minus_alphas_cumprod):
    B, C, H, W = x_start.shape
    grid = (B, C // CB)

    data_spec = pl.BlockSpec((1, CB, H, W), lambda b, c, *_: (b, c, 0, 0))
    tc_out = pl.pallas_call(
        _qsample_body,
        grid_spec=pltpu.PrefetchScalarGridSpec(
            num_scalar_prefetch=3,
            grid=grid,
            in_specs=[data_spec, data_spec],
            out_specs=data_spec,
        ),
        out_shape=jax.ShapeDtypeStruct((B, C, H, W), x_start.dtype),
        compiler_params=pltpu.CompilerParams(
            dimension_semantics=("parallel", "arbitrary"),
        ),
    )(t, sqrt_alphas_cumprod, sqrt_one_minus_alphas_cumprod, x_start, noise)

    sc_fn = _make_sc_probe(B, C, H, W, 2)
    sc_out = sc_fn(x_start, noise)
    return tc_out, sc_out
